# baseline (speedup 1.0000x reference)
import functools

import jax
import jax.numpy as jnp
from jax import lax
from jax.experimental import pallas as pl
from jax.experimental.pallas import tpu as pltpu
from jax.experimental.pallas import tpu_sc as plsc

B, L, D, P = 64, 5941, 32, 64
LPAD = 5952
NW = 32
N_REAL = B * P + B
N_PAD = 4352
B_PER_W = N_PAD // NW
CHUNKS = ((0, 128), (128, 8))
EF = B * L // 4
WF = LPAD // 4


_BM = 256


def _dw_body(d_ref, w_ref, b_ref, o_ref):
    o_ref[...] = (
        jnp.dot(
            d_ref[...].astype(jnp.bfloat16),
            w_ref[...].astype(jnp.bfloat16),
            preferred_element_type=jnp.float32,
        )
        + b_ref[...]
    )


def _compute_dw(distance_matrix, W, b):
    grid = (pl.cdiv(LPAD, _BM),)
    return pl.pallas_call(
        _dw_body,
        grid=grid,
        in_specs=[
            pl.BlockSpec((_BM, L), lambda i: (i, 0)),
            pl.BlockSpec((L, D), lambda i: (0, 0)),
            pl.BlockSpec((1, D), lambda i: (0, 0)),
        ],
        out_specs=pl.BlockSpec((_BM, D), lambda i: (i, 0)),
        out_shape=jax.ShapeDtypeStruct((LPAD, D), jnp.float32),
    )(distance_matrix, W, b.reshape(1, D))


def _sc_gather_body(tab_hbm, idx_hbm, out_hbm, idx_v, rows_v, sem):
    wid = lax.axis_index("s") * 2 + lax.axis_index("c")
    base = wid * B_PER_W
    pltpu.sync_copy(idx_hbm.at[pl.ds(base, B_PER_W)], idx_v)
    for off, cs in CHUNKS:
        pltpu.async_copy(
            tab_hbm.at[idx_v.at[pl.ds(off, cs)]],
            rows_v.at[pl.ds(off, cs)],
            sem,
        ).wait()
    pltpu.sync_copy(rows_v, out_hbm.at[pl.ds(base, B_PER_W)])


def _sc_gather(table, idx, width, linear):
    mesh = plsc.VectorSubcoreMesh(core_axis_name="c", subcore_axis_name="s")
    kern = functools.partial(
        pl.kernel,
        mesh=mesh,
        out_type=jax.ShapeDtypeStruct((N_PAD, width), jnp.float32),
        scratch_types=[
            pltpu.VMEM((B_PER_W,), jnp.int32),
            pltpu.VMEM((B_PER_W, width), jnp.float32),
            pltpu.SemaphoreType.DMA,
        ],
        compiler_params=(
            pltpu.CompilerParams(use_tc_tiling_on_sc=False) if linear else None
        ),
    )(_sc_gather_body)
    return kern(table, idx)


def _quarter(fat, q):
    out = fat[:, 0:D]
    for g in (1, 2, 3):
        out = jnp.where(q == g, fat[:, g * D : (g + 1) * D], out)
    return out


def _loop_body(pe_ref, pw_ref, de_ref, dw_ref, ql_ref, qdw_ref,
               out_ref, prb_ref, a_ref):
    pe = pe_ref[...]
    pw = _quarter(pw_ref[...], ql_ref[...])
    prb_ref[:, 0:D] = (pe * 0.05).astype(jnp.bfloat16)
    prb_ref[:, D : 2 * D] = (pw * 0.05).astype(jnp.bfloat16)

    def gram(bi, _):
        blk = prb_ref[pl.ds(bi * P, P), :]
        a_ref[pl.ds(bi * P, P), :] = lax.dot_general(
            blk, blk, (((1,), (1,)), ((), ())),
            preferred_element_type=jnp.float32,
        )
        return 0

    lax.fori_loop(0, B, gram, 0)

    de = de_ref[...]
    dw = _quarter(dw_ref[...], qdw_ref[...])
    drv = jnp.concatenate(
        [(de * 0.05).astype(jnp.bfloat16),
         (dw * 0.05).astype(jnp.bfloat16)], axis=1
    ).astype(jnp.float32)
    pr3 = prb_ref[...].astype(jnp.float32).reshape(B, P, 2 * D)
    attn0 = jnp.sum(pr3 * drv[:, None, :], axis=2)

    iota_p = lax.broadcasted_iota(jnp.int32, (B, P), 1)

    def step(t, carry):
        attn, mask = carry
        mx = jnp.max(attn, axis=1, keepdims=True)
        e = jnp.exp(attn - mx)
        out_ref[pl.ds(t, 1)] = (e / jnp.sum(e, axis=1, keepdims=True))[None]
        masked = attn * mask
        mmax = jnp.max(masked, axis=1, keepdims=True)
        cand = jnp.where(masked == mmax, iota_p, P)
        sel = jnp.min(cand, axis=1, keepdims=True)
        onehot = (iota_p == sel).astype(jnp.float32)
        mask = mask * (1.0 - onehot)
        a3 = a_ref[...].reshape(B, P, P)
        attn = jnp.sum(a3 * onehot[:, :, None], axis=1)
        return attn, mask

    lax.fori_loop(0, P, step, (attn0, jnp.ones((B, P), jnp.float32)))


def _pointer_loop(pe, pw, de, dw, ql, qdw):
    return pl.pallas_call(
        _loop_body,
        out_shape=jax.ShapeDtypeStruct((P, B, P), jnp.float32),
        scratch_shapes=[
            pltpu.VMEM((B * P, 2 * D), jnp.bfloat16),
            pltpu.VMEM((B * P, P), jnp.float32),
        ],
    )(pe, pw, de, dw, ql, qdw)


def kernel(edge_rep, distance_matrix, W, b, passenger_idx, driver_idx):
    p_flat = passenger_idx.reshape(-1).astype(jnp.int32)
    d_flat = driver_idx.astype(jnp.int32)
    pad = jnp.zeros((N_PAD - N_REAL,), jnp.int32)
    lidx = jnp.concatenate([p_flat, d_flat, pad])
    case_off = (jnp.arange(B, dtype=jnp.int32) * L)
    goff = jnp.concatenate([jnp.repeat(case_off, P), case_off, pad])
    gidx = lidx + goff

    edge_flat = edge_rep.reshape(B * L, D)
    ge = _sc_gather(edge_flat, gidx, D, linear=True)
    dwt = _compute_dw(distance_matrix, W, b)
    gw = _sc_gather(dwt.reshape(WF, 4 * D), lidx // 4, 4 * D, linear=False)

    ql = (lidx % 4).reshape(N_PAD, 1)
    out = _pointer_loop(
        ge[: B * P], gw[: B * P], ge[B * P : N_REAL], gw[B * P : N_REAL],
        ql[: B * P], ql[B * P : N_REAL],
    )
    return jnp.transpose(out, (1, 0, 2))


# speedup vs baseline: 1.8308x; 1.8308x over previous
import functools

import jax
import jax.numpy as jnp
from jax import lax
from jax.experimental import pallas as pl
from jax.experimental.pallas import tpu as pltpu
from jax.experimental.pallas import tpu_sc as plsc

B, L, D, P = 64, 5941, 32, 64
LPAD = 5952
PX = 72
NW = 32
N_REAL = B * P + B
N_PAD = 4352
B_PER_W = N_PAD // NW
CHUNKS = ((0, 128), (128, 8))
WF = LPAD // 4


_BM = 256


def _dw_body(d_ref, w_ref, b_ref, o_ref):
    o_ref[...] = (
        jnp.dot(
            d_ref[...].astype(jnp.bfloat16),
            w_ref[...].astype(jnp.bfloat16),
            preferred_element_type=jnp.float32,
        )
        + b_ref[...]
    )


def _compute_dw(distance_matrix, W, b):
    grid = (pl.cdiv(LPAD, _BM),)
    return pl.pallas_call(
        _dw_body,
        grid=grid,
        in_specs=[
            pl.BlockSpec((_BM, L), lambda i: (i, 0)),
            pl.BlockSpec((L, D), lambda i: (0, 0)),
            pl.BlockSpec((1, D), lambda i: (0, 0)),
        ],
        out_specs=pl.BlockSpec((_BM, D), lambda i: (i, 0)),
        out_shape=jax.ShapeDtypeStruct((LPAD, D), jnp.float32),
    )(distance_matrix, W, b.reshape(1, D))


def _eg_body(etv_ref, pidx_ref, out_ref):
    x = etv_ref[0]
    idx_row = pidx_ref[0]
    iota_l = lax.broadcasted_iota(jnp.int32, (L, PX), 0)
    oh = (iota_l == idx_row).astype(jnp.bfloat16)

    h1 = x.astype(jnp.bfloat16)
    r1 = x - h1.astype(jnp.float32)
    h2 = r1.astype(jnp.bfloat16)
    h3 = (r1 - h2.astype(jnp.float32)).astype(jnp.bfloat16)

    def sel(h):
        return lax.dot_general(h, oh, (((1,), (0,)), ((), ())),
                               preferred_element_type=jnp.float32)

    out_ref[0] = sel(h1) + sel(h2) + sel(h3)


def _edge_gather(etv, pidx_ext):
    return pl.pallas_call(
        _eg_body,
        grid=(B,),
        in_specs=[
            pl.BlockSpec((1, D, L), lambda b: (b, 0, 0)),
            pl.BlockSpec((1, 1, PX), lambda b: (b, 0, 0)),
        ],
        out_specs=pl.BlockSpec((1, D, PX), lambda b: (b, 0, 0)),
        out_shape=jax.ShapeDtypeStruct((B, D, PX), jnp.float32),
    )(etv, pidx_ext)


def _sc_gather_body(tab_hbm, idx_hbm, out_hbm, idx_v, rows_v, sem):
    wid = lax.axis_index("s") * 2 + lax.axis_index("c")
    base = wid * B_PER_W
    pltpu.sync_copy(idx_hbm.at[pl.ds(base, B_PER_W)], idx_v)
    for off, cs in CHUNKS:
        pltpu.async_copy(
            tab_hbm.at[idx_v.at[pl.ds(off, cs)]],
            rows_v.at[pl.ds(off, cs)],
            sem,
        ).wait()
    pltpu.sync_copy(rows_v, out_hbm.at[pl.ds(base, B_PER_W)])


def _sc_gather(table_fat, idx):
    mesh = plsc.VectorSubcoreMesh(core_axis_name="c", subcore_axis_name="s")
    kern = functools.partial(
        pl.kernel,
        mesh=mesh,
        out_type=jax.ShapeDtypeStruct((N_PAD, 4 * D), jnp.float32),
        scratch_types=[
            pltpu.VMEM((B_PER_W,), jnp.int32),
            pltpu.VMEM((B_PER_W, 4 * D), jnp.float32),
            pltpu.SemaphoreType.DMA,
        ],
    )(_sc_gather_body)
    return kern(table_fat, idx)


def _quarter(fat, q):
    out = fat[:, 0:D]
    for g in (1, 2, 3):
        out = jnp.where(q == g, fat[:, g * D : (g + 1) * D], out)
    return out


def _loop_body(peT_ref, pw_ref, dw_ref, ql_ref, qdw_ref, out_ref,
               pwb_ref, dwb_ref, a_ref, a0_ref):
    pwb_ref[...] = (_quarter(pw_ref[...], ql_ref[...]) * 0.05
                    ).astype(jnp.bfloat16)
    dwb_ref[...] = (_quarter(dw_ref[...], qdw_ref[...]) * 0.05
                    ).astype(jnp.bfloat16).astype(jnp.float32)

    def gram(bi, _):
        blk = peT_ref[pl.ds(bi, 1)][0]
        ebf = (blk[:, 0:P] * 0.05).astype(jnp.bfloat16)
        debf = (blk[:, P : P + 1] * 0.05).astype(jnp.bfloat16)
        wbk = pwb_ref[pl.ds(bi * P, P), :]
        dwk = dwb_ref[pl.ds(bi, 1), :].astype(jnp.bfloat16)
        a_e = lax.dot_general(ebf, ebf, (((0,), (0,)), ((), ())),
                              preferred_element_type=jnp.float32)
        a_w = lax.dot_general(wbk, wbk, (((1,), (1,)), ((), ())),
                              preferred_element_type=jnp.float32)
        a_ref[pl.ds(bi * P, P), :] = a_e + a_w
        a0_e = lax.dot_general(debf, ebf, (((0,), (0,)), ((), ())),
                               preferred_element_type=jnp.float32)
        a0_w = lax.dot_general(dwk, wbk, (((1,), (1,)), ((), ())),
                               preferred_element_type=jnp.float32)
        a0_ref[pl.ds(bi, 1), :] = a0_e + a0_w
        return 0

    lax.fori_loop(0, B, gram, 0)

    iota_p = lax.broadcasted_iota(jnp.int32, (B, P), 1)

    def step(t, carry):
        attn, mask = carry
        mx = jnp.max(attn, axis=1, keepdims=True)
        e = jnp.exp(attn - mx)
        out_ref[pl.ds(t, 1)] = (e / jnp.sum(e, axis=1, keepdims=True))[None]
        masked = attn * mask
        mmax = jnp.max(masked, axis=1, keepdims=True)
        cand = jnp.where(masked == mmax, iota_p, P)
        sel = jnp.min(cand, axis=1, keepdims=True)
        onehot = (iota_p == sel).astype(jnp.float32)
        mask = mask * (1.0 - onehot)
        a3 = a_ref[...].reshape(B, P, P)
        attn = jnp.sum(a3 * onehot[:, :, None], axis=1)
        return attn, mask

    lax.fori_loop(0, P, step, (a0_ref[...], jnp.ones((B, P), jnp.float32)))


def _pointer_loop(peT, pw, dw, ql, qdw):
    return pl.pallas_call(
        _loop_body,
        out_shape=jax.ShapeDtypeStruct((P, B, P), jnp.float32),
        scratch_shapes=[
            pltpu.VMEM((B * P, D), jnp.bfloat16),
            pltpu.VMEM((B, D), jnp.float32),
            pltpu.VMEM((B * P, P), jnp.float32),
            pltpu.VMEM((B, P), jnp.float32),
        ],
    )(peT, pw, dw, ql, qdw)


def kernel(edge_rep, distance_matrix, W, b, passenger_idx, driver_idx):
    p_idx = passenger_idx.astype(jnp.int32)
    d_idx = driver_idx.astype(jnp.int32)
    pidx_ext = jnp.concatenate(
        [p_idx, d_idx[:, None],
         jnp.zeros((B, PX - P - 1), jnp.int32)], axis=1
    ).reshape(B, 1, PX)

    etv = jnp.transpose(edge_rep, (0, 2, 1))
    peT = _edge_gather(etv, pidx_ext)

    dwt = _compute_dw(distance_matrix, W, b)

    p_flat = p_idx.reshape(-1)
    pad = jnp.zeros((N_PAD - N_REAL,), jnp.int32)
    lidx = jnp.concatenate([p_flat, d_idx, pad])
    gw = _sc_gather(dwt.reshape(WF, 4 * D), lidx // 4)

    ql = (lidx % 4).reshape(N_PAD, 1)
    out = _pointer_loop(peT, gw[: B * P], gw[B * P : N_REAL],
                        ql[: B * P], ql[B * P : N_REAL])
    return jnp.transpose(out, (1, 0, 2))


# speedup vs baseline: 2.0350x; 1.1115x over previous
import functools

import jax
import jax.numpy as jnp
from jax import lax
from jax.experimental import pallas as pl
from jax.experimental.pallas import tpu as pltpu
from jax.experimental.pallas import tpu_sc as plsc

B, L, D, P = 64, 5941, 32, 64
PX = 72
ROWS_PER = 96
LPAD = B * ROWS_PER
NW = 32
N_REAL = B * P + B
N_PAD = 4352
B_PER_W = N_PAD // NW
CHUNKS = ((0, 128), (128, 8))
WF = LPAD // 4


def _fused_body(d_ref, w_ref, b_ref, etv_ref, pidx_ref, dw_ref, pe_ref):
    dw_ref[...] = (
        jnp.dot(
            d_ref[...].astype(jnp.bfloat16),
            w_ref[...].astype(jnp.bfloat16),
            preferred_element_type=jnp.float32,
        )
        + b_ref[...]
    )

    idx_row = pidx_ref[0]
    iota_l = lax.broadcasted_iota(jnp.int32, (L, PX), 0)
    oh = (iota_l == idx_row).astype(jnp.bfloat16)
    h = (etv_ref[0] * 0.05).astype(jnp.bfloat16)
    pe_ref[0] = lax.dot_general(h, oh, (((1,), (0,)), ((), ())),
                                preferred_element_type=jnp.float32)


def _fused_dw_egather(distance_matrix, W, b, etv, pidx_ext):
    return pl.pallas_call(
        _fused_body,
        grid=(B,),
        in_specs=[
            pl.BlockSpec((ROWS_PER, L),
                         lambda i: (jnp.minimum(i, (L - 1) // ROWS_PER), 0)),
            pl.BlockSpec((L, D), lambda i: (0, 0)),
            pl.BlockSpec((1, D), lambda i: (0, 0)),
            pl.BlockSpec((1, D, L), lambda i: (i, 0, 0)),
            pl.BlockSpec((1, 1, PX), lambda i: (i, 0, 0)),
        ],
        out_specs=[
            pl.BlockSpec((ROWS_PER, D), lambda i: (i, 0)),
            pl.BlockSpec((1, D, PX), lambda i: (i, 0, 0)),
        ],
        out_shape=[
            jax.ShapeDtypeStruct((LPAD, D), jnp.float32),
            jax.ShapeDtypeStruct((B, D, PX), jnp.float32),
        ],
    )(distance_matrix, W, b.reshape(1, D), etv, pidx_ext)


def _sc_gather_body(tab_hbm, idx_hbm, out_hbm, idx_v, rows_v, sem):
    wid = lax.axis_index("s") * 2 + lax.axis_index("c")
    base = wid * B_PER_W
    pltpu.sync_copy(idx_hbm.at[pl.ds(base, B_PER_W)], idx_v)
    for off, cs in CHUNKS:
        pltpu.async_copy(
            tab_hbm.at[idx_v.at[pl.ds(off, cs)]],
            rows_v.at[pl.ds(off, cs)],
            sem,
        ).wait()
    pltpu.sync_copy(rows_v, out_hbm.at[pl.ds(base, B_PER_W)])


def _sc_gather(table_fat, idx):
    mesh = plsc.VectorSubcoreMesh(core_axis_name="c", subcore_axis_name="s")
    kern = functools.partial(
        pl.kernel,
        mesh=mesh,
        out_type=jax.ShapeDtypeStruct((N_PAD, 4 * D), jnp.float32),
        scratch_types=[
            pltpu.VMEM((B_PER_W,), jnp.int32),
            pltpu.VMEM((B_PER_W, 4 * D), jnp.float32),
            pltpu.SemaphoreType.DMA,
        ],
    )(_sc_gather_body)
    return kern(table_fat, idx)


def _quarter(fat, q):
    out = fat[:, 0:D]
    for g in (1, 2, 3):
        out = jnp.where(q == g, fat[:, g * D : (g + 1) * D], out)
    return out


def _loop_body(peT_ref, pw_ref, dw_ref, ql_ref, qdw_ref, out_ref,
               pwb_ref, dwb_ref, at_ref, a0_ref):
    pwb_ref[...] = (_quarter(pw_ref[...], ql_ref[...]) * 0.05
                    ).astype(jnp.bfloat16)
    dwb_ref[...] = (_quarter(dw_ref[...], qdw_ref[...]) * 0.05
                    ).astype(jnp.bfloat16).astype(jnp.float32)

    def gram(bi, _):
        blk = peT_ref[pl.ds(bi, 1)][0]
        ebf = blk[:, 0:P].astype(jnp.bfloat16)
        debf = blk[:, P : P + 1].astype(jnp.bfloat16)
        wbk = pwb_ref[pl.ds(bi * P, P), :]
        dwk = dwb_ref[pl.ds(bi, 1), :].astype(jnp.bfloat16)
        a_e = lax.dot_general(ebf, ebf, (((0,), (0,)), ((), ())),
                              preferred_element_type=jnp.float32)
        a_w = lax.dot_general(wbk, wbk, (((1,), (1,)), ((), ())),
                              preferred_element_type=jnp.float32)
        at_ref[:, pl.ds(bi, 1), :] = (a_e + a_w)[:, None, :]
        a0_e = lax.dot_general(debf, ebf, (((0,), (0,)), ((), ())),
                               preferred_element_type=jnp.float32)
        a0_w = lax.dot_general(dwk, wbk, (((1,), (1,)), ((), ())),
                               preferred_element_type=jnp.float32)
        a0_ref[pl.ds(bi, 1), :] = a0_e + a0_w
        return 0

    lax.fori_loop(0, B, gram, 0)

    iota_p = lax.broadcasted_iota(jnp.int32, (B, P), 1)

    def step(t, carry):
        attn, mask = carry
        mx = jnp.max(attn, axis=1, keepdims=True)
        e = jnp.exp(attn - mx)
        out_ref[pl.ds(t, 1)] = (e / jnp.sum(e, axis=1, keepdims=True))[None]
        masked = attn * mask
        mmax = jnp.max(masked, axis=1, keepdims=True)
        cand = jnp.where(masked == mmax, iota_p, P)
        sel = jnp.min(cand, axis=1, keepdims=True)
        onehot = (iota_p == sel).astype(jnp.float32)
        mask = mask * (1.0 - onehot)
        oh2 = jnp.swapaxes(onehot, 0, 1)
        attn = jnp.sum(at_ref[...] * oh2[:, :, None], axis=0)
        return attn, mask

    lax.fori_loop(0, P, step, (a0_ref[...], jnp.ones((B, P), jnp.float32)))


def _pointer_loop(peT, pw, dw, ql, qdw):
    return pl.pallas_call(
        _loop_body,
        out_shape=jax.ShapeDtypeStruct((P, B, P), jnp.float32),
        scratch_shapes=[
            pltpu.VMEM((B * P, D), jnp.bfloat16),
            pltpu.VMEM((B, D), jnp.float32),
            pltpu.VMEM((P, B, P), jnp.float32),
            pltpu.VMEM((B, P), jnp.float32),
        ],
    )(peT, pw, dw, ql, qdw)


def kernel(edge_rep, distance_matrix, W, b, passenger_idx, driver_idx):
    p_idx = passenger_idx.astype(jnp.int32)
    d_idx = driver_idx.astype(jnp.int32)
    pidx_ext = jnp.concatenate(
        [p_idx, d_idx[:, None],
         jnp.zeros((B, PX - P - 1), jnp.int32)], axis=1
    ).reshape(B, 1, PX)

    etv = jnp.transpose(edge_rep, (0, 2, 1))
    dwt, peT = _fused_dw_egather(distance_matrix, W, b, etv, pidx_ext)

    p_flat = p_idx.reshape(-1)
    pad = jnp.zeros((N_PAD - N_REAL,), jnp.int32)
    lidx = jnp.concatenate([p_flat, d_idx, pad])
    gw = _sc_gather(dwt.reshape(WF, 4 * D), lidx // 4)

    ql = (lidx % 4).reshape(N_PAD, 1)
    out = _pointer_loop(peT, gw[: B * P], gw[B * P : N_REAL],
                        ql[: B * P], ql[B * P : N_REAL])
    return jnp.transpose(out, (1, 0, 2))
